# baseline (speedup 1.0000x reference)
import functools

import jax
import jax.numpy as jnp
from jax import lax
from jax.experimental import pallas as pl
from jax.experimental.pallas import tpu as pltpu
from jax.experimental.pallas import tpu_sc as plsc

N = 16384
M = 4096
B = 4
C = 64
MID = 16
ROWS = 1024
NT = N // ROWS
NW = 32
RPW = N // NW
LANES = 16


CHUNK = 512


def _argmin_body(hc_ref, lc_ref, idx_ref):
    hc = hc_ref[...]
    lc = lc_ref[...]
    ln = jnp.sum(lc * lc, axis=1, keepdims=True)
    hc4 = jnp.concatenate([hc, jnp.ones((ROWS, 1), jnp.float32)], axis=1)
    lc4 = jnp.concatenate([lc, -0.5 * ln], axis=1)
    g = lax.dot_general(hc4, lc4, (((1,), (1,)), ((), ())),
                        preferred_element_type=jnp.float32)
    run_max = g[:, :CHUNK]
    run_k = jnp.zeros((ROWS, CHUNK), jnp.int32)
    for k in range(1, M // CHUNK):
        v = g[:, k * CHUNK:(k + 1) * CHUNK]
        pred = v > run_max
        run_max = jnp.maximum(run_max, v)
        run_k = jnp.where(pred, k, run_k)
    rowmax = jnp.max(run_max, axis=1, keepdims=True)
    pos = lax.broadcasted_iota(jnp.int32, (ROWS, CHUNK), 1)
    cand = run_k * CHUNK + pos
    idx = jnp.min(jnp.where(run_max == rowmax, cand, M), axis=1,
                  keepdims=True)
    idx_ref[...] = idx


def _nn_indices(high_res_coord, low_res_coord):
    return pl.pallas_call(
        _argmin_body,
        grid=(NT,),
        in_specs=[
            pl.BlockSpec((ROWS, 3), lambda i: (i, 0)),
            pl.BlockSpec((M, 3), lambda i: (0, 0)),
        ],
        out_specs=pl.BlockSpec((ROWS, 1), lambda i: (i, 0)),
        out_shape=jax.ShapeDtypeStruct((N, 1), jnp.int32),
    )(high_res_coord, low_res_coord)


def _hist_body(idx_hbm, out_hbm, idx_v, hist_v):
    c = lax.axis_index("c")
    s = lax.axis_index("s")
    wid = s * 2 + c
    base = wid * RPW
    pltpu.sync_copy(idx_hbm.at[pl.ds(base, RPW)], idx_v)
    zeros16 = jnp.zeros((LANES,), jnp.float32)

    def zero_body(i, carry):
        hist_v[pl.ds(i * LANES, LANES)] = zeros16
        return carry

    lax.fori_loop(0, M // LANES, zero_body, 0)

    ones16 = jnp.ones((LANES,), jnp.float32)
    lane_iota = lax.iota(jnp.int32, LANES)

    def scat_body(j, carry):
        idx16 = idx_v[pl.ds(j * LANES, LANES)]
        for l in range(LANES):
            plsc.addupdate_scatter(hist_v, [idx16], ones16,
                                   mask=lane_iota == l)
        return carry

    lax.fori_loop(0, RPW // LANES, scat_body, 0)
    pltpu.sync_copy(hist_v, out_hbm.at[wid])


def _partial_hist(idx_flat):
    mesh = plsc.VectorSubcoreMesh(core_axis_name="c", subcore_axis_name="s")
    return pl.kernel(
        _hist_body,
        out_type=jax.ShapeDtypeStruct((NW, M), jnp.float32),
        scratch_types=[
            pltpu.VMEM((RPW,), jnp.int32),
            pltpu.VMEM((M,), jnp.float32),
        ],
        mesh=mesh,
        compiler_params=pltpu.CompilerParams(needs_layout_passes=False),
    )(idx_flat)


FROWS = 2048
FNT = N // FROWS


def _fuse_body(ph_ref, lr_ref, hf_ref, hco_ref, off_ref,
               caw1_ref, cab1_ref, caw2_ref, cab2_ref,
               saw1_ref, sab1_ref, saw2_ref, sab2_ref,
               fmw_ref, fmb_ref, gam_ref, bet_ref, out_ref,
               y_s, cw_s, st_s):
    f32 = jnp.float32
    p = pl.program_id(0)
    t = pl.program_id(1)

    @pl.when(p == 0)
    def _phase0():
        @pl.when(t == 0)
        def _prep():
            ph = ph_ref[...]
            wpb = NW // B
            hist = jnp.concatenate(
                [jnp.sum(ph[wpb * b:wpb * (b + 1)], axis=0, keepdims=True)
                 for b in range(B)], axis=0)
            counts = jnp.sum(hist, axis=1, keepdims=True)
            gf = lax.dot_general(hist, lr_ref[...], (((1,), (0,)), ((), ())),
                                 preferred_element_type=f32)
            gf = gf / jnp.maximum(counts, 1.0)
            h = jnp.maximum(jnp.dot(gf, caw1_ref[...]) + cab1_ref[...], 0.0)
            cw_s[0:B] = jax.nn.sigmoid(jnp.dot(h, caw2_ref[...])
                                       + cab2_ref[...])
            st_s[...] = jnp.zeros((8, C), f32)

        cw = cw_s[0:B]
        ii = (t * FROWS
              + lax.broadcasted_iota(jnp.int32, (FROWS, B), 0))
        bidx = jnp.sum((ii >= off_ref[...]).astype(jnp.int32), axis=1,
                       keepdims=True)
        oh = (bidx == lax.broadcasted_iota(jnp.int32, (FROWS, B), 1))
        cwr = jnp.dot(oh.astype(f32), cw, preferred_element_type=f32)
        hf = hf_ref[...]
        cr = hf * cwr
        saw1 = saw1_ref[...]
        s1 = (jnp.dot(cr, saw1[:C]) + jnp.dot(hco_ref[...], saw1[C:])
              + sab1_ref[...])
        s1 = jnp.maximum(s1, 0.0)
        sw = jax.nn.sigmoid(jnp.dot(s1, saw2_ref[...]) + sab2_ref[...])
        sr = cr * sw
        fmw = fmw_ref[...]
        y = jnp.dot(hf, fmw[:C]) + jnp.dot(sr, fmw[C:]) + fmb_ref[...]
        y_s[pl.ds(t * FROWS, FROWS), :] = y
        st_s[0:1] += jnp.sum(y, axis=0, keepdims=True)
        st_s[1:2] += jnp.sum(y * y, axis=0, keepdims=True)

        @pl.when(t == FNT - 1)
        def _stats():
            mean = st_s[0:1] * (1.0 / N)
            var = st_s[1:2] * (1.0 / N) - mean * mean
            st_s[2:3] = mean
            st_s[3:4] = lax.rsqrt(var + 1e-5)

    @pl.when(p == 1)
    def _phase1():
        y = y_s[pl.ds(t * FROWS, FROWS), :]
        out_ref[...] = jnp.maximum(
            (y - st_s[2:3]) * st_s[3:4] * gam_ref[...] + bet_ref[...], 0.0)


def _fuse(ph, lr, hf, hco, off2d, caw1, cab1, caw2, cab2,
          saw1, sab1, saw2, sab2, fmw, fmb, gam, bet):
    full = lambda shape: pl.BlockSpec(shape, lambda p, t: tuple(
        0 for _ in shape))
    return pl.pallas_call(
        _fuse_body,
        grid=(2, FNT),
        in_specs=[
            full((NW, M)),
            full((M, C)),
            pl.BlockSpec((FROWS, C), lambda p, t: ((1 - p) * t, 0)),
            pl.BlockSpec((FROWS, 3), lambda p, t: ((1 - p) * t, 0)),
            full((1, B)),
            full((C, MID)), full((1, MID)), full((MID, C)), full((1, C)),
            full((C + 3, C)), full((1, C)), full((C, 1)), full((1, 1)),
            full((2 * C, C)), full((1, C)), full((1, C)), full((1, C)),
        ],
        out_specs=pl.BlockSpec((FROWS, C), lambda p, t: (p * t, 0)),
        out_shape=jax.ShapeDtypeStruct((N, C), jnp.float32),
        scratch_shapes=[
            pltpu.VMEM((N, C), jnp.float32),
            pltpu.VMEM((8, C), jnp.float32),
            pltpu.VMEM((8, C), jnp.float32),
        ],
    )(ph, lr, hf, hco, off2d, caw1, cab1, caw2, cab2,
      saw1, sab1, saw2, sab2, fmw, fmb, gam, bet)


@jax.jit
def kernel(high_res_feat, high_res_coord, low_res_feat, low_res_coord,
           high_res_offset, ca_w1, ca_b1, ca_w2, ca_b2,
           sa_w1, sa_b1, sa_w2, sa_b2, fm_w, fm_b, bn_gamma, bn_beta):
    idx = _nn_indices(high_res_coord, low_res_coord)
    ph = _partial_hist(idx.reshape(N))
    ph = _partial_hist(idx.reshape(N))
    return _fuse(
        ph, low_res_feat, high_res_feat, high_res_coord,
        high_res_offset.reshape(1, B),
        ca_w1, ca_b1.reshape(1, -1), ca_w2, ca_b2.reshape(1, -1),
        sa_w1, sa_b1.reshape(1, -1), sa_w2, sa_b2.reshape(1, -1),
        fm_w, fm_b.reshape(1, -1), bn_gamma.reshape(1, -1),
        bn_beta.reshape(1, -1))


# speedup vs baseline: 1.1807x; 1.1807x over previous
import functools

import jax
import jax.numpy as jnp
from jax import lax
from jax.experimental import pallas as pl
from jax.experimental.pallas import tpu as pltpu
from jax.experimental.pallas import tpu_sc as plsc

N = 16384
M = 4096
B = 4
C = 64
MID = 16
ROWS = 1024
NT = N // ROWS
NW = 32
RPW = N // NW
LANES = 16


CHUNK = 512


def _argmin_body(hc_ref, lct_ref, idx_ref):
    hc = hc_ref[...]
    lct = lct_ref[...]
    ln = jnp.sum(lct * lct, axis=0, keepdims=True)
    hc4 = jnp.concatenate([hc, jnp.ones((ROWS, 1), jnp.float32)], axis=1)
    lct4 = jnp.concatenate([lct, -0.5 * ln], axis=0)
    g = lax.dot_general(hc4, lct4, (((1,), (0,)), ((), ())),
                        preferred_element_type=jnp.float32)
    idx = jnp.argmax(g, axis=1).astype(jnp.int32)
    idx_ref[...] = idx.reshape(ROWS, 1)


def _nn_indices(high_res_coord, lct):
    return pl.pallas_call(
        _argmin_body,
        grid=(NT,),
        in_specs=[
            pl.BlockSpec((ROWS, 3), lambda i: (i, 0)),
            pl.BlockSpec((3, M), lambda i: (0, 0)),
        ],
        out_specs=pl.BlockSpec((ROWS, 1), lambda i: (i, 0)),
        out_shape=jax.ShapeDtypeStruct((N, 1), jnp.int32),
    )(high_res_coord, lct)


def _hist_body(idx_hbm, out_hbm, idx_v, hist_v):
    c = lax.axis_index("c")
    s = lax.axis_index("s")
    wid = s * 2 + c
    base = wid * RPW
    pltpu.sync_copy(idx_hbm.at[pl.ds(base, RPW)], idx_v)
    zeros16 = jnp.zeros((LANES,), jnp.float32)

    def zero_body(i, carry):
        hist_v[pl.ds(i * LANES, LANES)] = zeros16
        return carry

    lax.fori_loop(0, M // LANES, zero_body, 0)

    ones16 = jnp.ones((LANES,), jnp.float32)
    lane_iota = lax.iota(jnp.int32, LANES)

    def scat_body(j, carry):
        idx16 = idx_v[pl.ds(j * LANES, LANES)]
        for l in range(LANES):
            plsc.addupdate_scatter(hist_v, [idx16], ones16,
                                   mask=lane_iota == l)
        return carry

    lax.fori_loop(0, RPW // LANES, scat_body, 0)
    pltpu.sync_copy(hist_v, out_hbm.at[wid])


def _partial_hist(idx_flat):
    mesh = plsc.VectorSubcoreMesh(core_axis_name="c", subcore_axis_name="s")
    return pl.kernel(
        _hist_body,
        out_type=jax.ShapeDtypeStruct((NW, M), jnp.float32),
        scratch_types=[
            pltpu.VMEM((RPW,), jnp.int32),
            pltpu.VMEM((M,), jnp.float32),
        ],
        mesh=mesh,
        compiler_params=pltpu.CompilerParams(needs_layout_passes=False),
    )(idx_flat)


FROWS = 2048
FNT = N // FROWS


def _fuse_body(ph_ref, lr_ref, hf_ref, hco_ref, off_ref,
               caw1_ref, cab1_ref, caw2_ref, cab2_ref,
               saw1_ref, sab1_ref, saw2_ref, sab2_ref,
               fmw_ref, fmb_ref, gam_ref, bet_ref, out_ref,
               y_s, cw_s, st_s):
    f32 = jnp.float32
    p = pl.program_id(0)
    t = pl.program_id(1)

    @pl.when(p == 0)
    def _phase0():
        @pl.when(t == 0)
        def _prep():
            ph = ph_ref[...]
            wpb = NW // B
            hist = jnp.concatenate(
                [jnp.sum(ph[wpb * b:wpb * (b + 1)], axis=0, keepdims=True)
                 for b in range(B)], axis=0)
            counts = jnp.sum(hist, axis=1, keepdims=True)
            gf = lax.dot_general(hist, lr_ref[...], (((1,), (0,)), ((), ())),
                                 preferred_element_type=f32)
            gf = gf / jnp.maximum(counts, 1.0)
            h = jnp.maximum(jnp.dot(gf, caw1_ref[...]) + cab1_ref[...], 0.0)
            cw_s[0:B] = jax.nn.sigmoid(jnp.dot(h, caw2_ref[...])
                                       + cab2_ref[...])
            st_s[...] = jnp.zeros((8, C), f32)

        cw = cw_s[0:B]
        ii = (t * FROWS
              + lax.broadcasted_iota(jnp.int32, (FROWS, B), 0))
        bidx = jnp.sum((ii >= off_ref[...]).astype(jnp.int32), axis=1,
                       keepdims=True)
        oh = (bidx == lax.broadcasted_iota(jnp.int32, (FROWS, B), 1))
        cwr = jnp.dot(oh.astype(f32), cw, preferred_element_type=f32)
        hf = hf_ref[...]
        cr = hf * cwr
        saw1 = saw1_ref[...]
        s1 = (jnp.dot(cr, saw1[:C]) + jnp.dot(hco_ref[...], saw1[C:])
              + sab1_ref[...])
        s1 = jnp.maximum(s1, 0.0)
        sw = jax.nn.sigmoid(jnp.dot(s1, saw2_ref[...]) + sab2_ref[...])
        sr = cr * sw
        fmw = fmw_ref[...]
        y = jnp.dot(hf, fmw[:C]) + jnp.dot(sr, fmw[C:]) + fmb_ref[...]
        y_s[pl.ds(t * FROWS, FROWS), :] = y
        st_s[0:1] += jnp.sum(y, axis=0, keepdims=True)
        st_s[1:2] += jnp.sum(y * y, axis=0, keepdims=True)

        @pl.when(t == FNT - 1)
        def _stats():
            mean = st_s[0:1] * (1.0 / N)
            var = st_s[1:2] * (1.0 / N) - mean * mean
            st_s[2:3] = mean
            st_s[3:4] = lax.rsqrt(var + 1e-5)

    @pl.when(p == 1)
    def _phase1():
        y = y_s[pl.ds(t * FROWS, FROWS), :]
        out_ref[...] = jnp.maximum(
            (y - st_s[2:3]) * st_s[3:4] * gam_ref[...] + bet_ref[...], 0.0)


def _fuse(ph, lr, hf, hco, off2d, caw1, cab1, caw2, cab2,
          saw1, sab1, saw2, sab2, fmw, fmb, gam, bet):
    full = lambda shape: pl.BlockSpec(shape, lambda p, t: tuple(
        0 for _ in shape))
    return pl.pallas_call(
        _fuse_body,
        grid=(2, FNT),
        in_specs=[
            full((NW, M)),
            full((M, C)),
            pl.BlockSpec((FROWS, C), lambda p, t: ((1 - p) * t, 0)),
            pl.BlockSpec((FROWS, 3), lambda p, t: ((1 - p) * t, 0)),
            full((1, B)),
            full((C, MID)), full((1, MID)), full((MID, C)), full((1, C)),
            full((C + 3, C)), full((1, C)), full((C, 1)), full((1, 1)),
            full((2 * C, C)), full((1, C)), full((1, C)), full((1, C)),
        ],
        out_specs=pl.BlockSpec((FROWS, C), lambda p, t: (p * t, 0)),
        out_shape=jax.ShapeDtypeStruct((N, C), jnp.float32),
        scratch_shapes=[
            pltpu.VMEM((N, C), jnp.float32),
            pltpu.VMEM((8, C), jnp.float32),
            pltpu.VMEM((8, C), jnp.float32),
        ],
    )(ph, lr, hf, hco, off2d, caw1, cab1, caw2, cab2,
      saw1, sab1, saw2, sab2, fmw, fmb, gam, bet)


@jax.jit
def kernel(high_res_feat, high_res_coord, low_res_feat, low_res_coord,
           high_res_offset, ca_w1, ca_b1, ca_w2, ca_b2,
           sa_w1, sa_b1, sa_w2, sa_b2, fm_w, fm_b, bn_gamma, bn_beta):
    lct = low_res_coord.T
    idx = _nn_indices(high_res_coord, lct)
    ph = _partial_hist(idx.reshape(N))
    ph = _partial_hist(idx.reshape(N))
    return _fuse(
        ph, low_res_feat, high_res_feat, high_res_coord,
        high_res_offset.reshape(1, B),
        ca_w1, ca_b1.reshape(1, -1), ca_w2, ca_b2.reshape(1, -1),
        sa_w1, sa_b1.reshape(1, -1), sa_w2, sa_b2.reshape(1, -1),
        fm_w, fm_b.reshape(1, -1), bn_gamma.reshape(1, -1),
        bn_beta.reshape(1, -1))


# speedup vs baseline: 1.2059x; 1.0214x over previous
import functools

import jax
import jax.numpy as jnp
from jax import lax
from jax.experimental import pallas as pl
from jax.experimental.pallas import tpu as pltpu
from jax.experimental.pallas import tpu_sc as plsc

N = 16384
M = 4096
B = 4
C = 64
MID = 16
ROWS = 2048
NT = N // ROWS
NW = 32
RPW = N // NW
LANES = 16


CHUNK = 512


def _argmin_body(hc_ref, lct_ref, idx_ref):
    hc = hc_ref[...]
    lct = lct_ref[...]
    ln = jnp.sum(lct * lct, axis=0, keepdims=True)
    hc4 = jnp.concatenate([hc, jnp.ones((ROWS, 1), jnp.float32)], axis=1)
    lct4 = jnp.concatenate([lct, -0.5 * ln], axis=0)
    g = lax.dot_general(hc4, lct4, (((1,), (0,)), ((), ())),
                        preferred_element_type=jnp.float32)
    idx = jnp.argmax(g, axis=1).astype(jnp.int32)
    idx_ref[...] = idx.reshape(ROWS, 1)


def _nn_indices(high_res_coord, lct):
    return pl.pallas_call(
        _argmin_body,
        grid=(NT,),
        in_specs=[
            pl.BlockSpec((ROWS, 3), lambda i: (i, 0)),
            pl.BlockSpec((3, M), lambda i: (0, 0)),
        ],
        out_specs=pl.BlockSpec((ROWS, 1), lambda i: (i, 0)),
        out_shape=jax.ShapeDtypeStruct((N, 1), jnp.int32),
    )(high_res_coord, lct)


def _hist_body(idx_hbm, out_hbm, idx_v, hist_v):
    c = lax.axis_index("c")
    s = lax.axis_index("s")
    wid = s * 2 + c
    base = wid * RPW
    pltpu.sync_copy(idx_hbm.at[pl.ds(base, RPW)], idx_v)
    zeros16 = jnp.zeros((LANES,), jnp.float32)

    def zero_body(i, carry):
        hist_v[pl.ds(i * LANES, LANES)] = zeros16
        return carry

    lax.fori_loop(0, M // LANES, zero_body, 0)

    ones16 = jnp.ones((LANES,), jnp.float32)
    lane_iota = lax.iota(jnp.int32, LANES)

    def scat_body(j, carry):
        idx16 = idx_v[pl.ds(j * LANES, LANES)]
        for l in range(LANES):
            plsc.addupdate_scatter(hist_v, [idx16], ones16,
                                   mask=lane_iota == l)
        return carry

    lax.fori_loop(0, RPW // LANES, scat_body, 0)
    pltpu.sync_copy(hist_v, out_hbm.at[wid])


def _partial_hist(idx_flat):
    mesh = plsc.VectorSubcoreMesh(core_axis_name="c", subcore_axis_name="s")
    return pl.kernel(
        _hist_body,
        out_type=jax.ShapeDtypeStruct((NW, M), jnp.float32),
        scratch_types=[
            pltpu.VMEM((RPW,), jnp.int32),
            pltpu.VMEM((M,), jnp.float32),
        ],
        mesh=mesh,
        compiler_params=pltpu.CompilerParams(needs_layout_passes=False),
    )(idx_flat)


FROWS = 2048
FNT = N // FROWS


def _fuse_body(ph_ref, lr_ref, hf_ref, hco_ref, off_ref,
               caw1_ref, cab1_ref, caw2_ref, cab2_ref,
               saw1_ref, sab1_ref, saw2_ref, sab2_ref,
               fmw_ref, fmb_ref, gam_ref, bet_ref, out_ref,
               y_s, cw_s, st_s):
    f32 = jnp.float32
    p = pl.program_id(0)
    t = pl.program_id(1)

    @pl.when(p == 0)
    def _phase0():
        @pl.when(t == 0)
        def _prep():
            ph = ph_ref[...]
            wpb = NW // B
            hist = jnp.concatenate(
                [jnp.sum(ph[wpb * b:wpb * (b + 1)], axis=0, keepdims=True)
                 for b in range(B)], axis=0)
            counts = jnp.sum(hist, axis=1, keepdims=True)
            gf = lax.dot_general(hist, lr_ref[...], (((1,), (0,)), ((), ())),
                                 preferred_element_type=f32)
            gf = gf / jnp.maximum(counts, 1.0)
            h = jnp.maximum(jnp.dot(gf, caw1_ref[...]) + cab1_ref[...], 0.0)
            cw_s[0:B] = jax.nn.sigmoid(jnp.dot(h, caw2_ref[...])
                                       + cab2_ref[...])
            st_s[...] = jnp.zeros((8, C), f32)

        cw = cw_s[0:B]
        ii = (t * FROWS
              + lax.broadcasted_iota(jnp.int32, (FROWS, B), 0))
        bidx = jnp.sum((ii >= off_ref[...]).astype(jnp.int32), axis=1,
                       keepdims=True)
        oh = (bidx == lax.broadcasted_iota(jnp.int32, (FROWS, B), 1))
        cwr = jnp.dot(oh.astype(f32), cw, preferred_element_type=f32)
        hf = hf_ref[...]
        cr = hf * cwr
        saw1 = saw1_ref[...]
        s1 = (jnp.dot(cr, saw1[:C]) + jnp.dot(hco_ref[...], saw1[C:])
              + sab1_ref[...])
        s1 = jnp.maximum(s1, 0.0)
        sw = jax.nn.sigmoid(jnp.dot(s1, saw2_ref[...]) + sab2_ref[...])
        sr = cr * sw
        fmw = fmw_ref[...]
        y = jnp.dot(hf, fmw[:C]) + jnp.dot(sr, fmw[C:]) + fmb_ref[...]
        y_s[pl.ds(t * FROWS, FROWS), :] = y
        st_s[0:1] += jnp.sum(y, axis=0, keepdims=True)
        st_s[1:2] += jnp.sum(y * y, axis=0, keepdims=True)

        @pl.when(t == FNT - 1)
        def _stats():
            mean = st_s[0:1] * (1.0 / N)
            var = st_s[1:2] * (1.0 / N) - mean * mean
            st_s[2:3] = mean
            st_s[3:4] = lax.rsqrt(var + 1e-5)

    @pl.when(p == 1)
    def _phase1():
        y = y_s[pl.ds(t * FROWS, FROWS), :]
        out_ref[...] = jnp.maximum(
            (y - st_s[2:3]) * st_s[3:4] * gam_ref[...] + bet_ref[...], 0.0)


def _fuse(ph, lr, hf, hco, off2d, caw1, cab1, caw2, cab2,
          saw1, sab1, saw2, sab2, fmw, fmb, gam, bet):
    full = lambda shape: pl.BlockSpec(shape, lambda p, t: tuple(
        0 for _ in shape))
    return pl.pallas_call(
        _fuse_body,
        grid=(2, FNT),
        in_specs=[
            full((NW, M)),
            full((M, C)),
            pl.BlockSpec((FROWS, C), lambda p, t: ((1 - p) * t, 0)),
            pl.BlockSpec((FROWS, 3), lambda p, t: ((1 - p) * t, 0)),
            full((1, B)),
            full((C, MID)), full((1, MID)), full((MID, C)), full((1, C)),
            full((C + 3, C)), full((1, C)), full((C, 1)), full((1, 1)),
            full((2 * C, C)), full((1, C)), full((1, C)), full((1, C)),
        ],
        out_specs=pl.BlockSpec((FROWS, C), lambda p, t: (p * t, 0)),
        out_shape=jax.ShapeDtypeStruct((N, C), jnp.float32),
        scratch_shapes=[
            pltpu.VMEM((N, C), jnp.float32),
            pltpu.VMEM((8, C), jnp.float32),
            pltpu.VMEM((8, C), jnp.float32),
        ],
    )(ph, lr, hf, hco, off2d, caw1, cab1, caw2, cab2,
      saw1, sab1, saw2, sab2, fmw, fmb, gam, bet)


@jax.jit
def kernel(high_res_feat, high_res_coord, low_res_feat, low_res_coord,
           high_res_offset, ca_w1, ca_b1, ca_w2, ca_b2,
           sa_w1, sa_b1, sa_w2, sa_b2, fm_w, fm_b, bn_gamma, bn_beta):
    lct = low_res_coord.T
    idx = _nn_indices(high_res_coord, lct)
    ph = _partial_hist(idx.reshape(N))
    ph = _partial_hist(idx.reshape(N))
    return _fuse(
        ph, low_res_feat, high_res_feat, high_res_coord,
        high_res_offset.reshape(1, B),
        ca_w1, ca_b1.reshape(1, -1), ca_w2, ca_b2.reshape(1, -1),
        sa_w1, sa_b1.reshape(1, -1), sa_w2, sa_b2.reshape(1, -1),
        fm_w, fm_b.reshape(1, -1), bn_gamma.reshape(1, -1),
        bn_beta.reshape(1, -1))


# speedup vs baseline: 1.2388x; 1.0272x over previous
import functools

import jax
import jax.numpy as jnp
from jax import lax
from jax.experimental import pallas as pl
from jax.experimental.pallas import tpu as pltpu
from jax.experimental.pallas import tpu_sc as plsc

N = 16384
M = 4096
B = 4
C = 64
MID = 16
ROWS = 2048
NT = N // ROWS
NW = 32
RPW = N // NW
LANES = 16


CHUNK = 512


def _argmin_body(hc_ref, lct_ref, idx_ref):
    hc = hc_ref[...]
    lct = lct_ref[...]
    ln = jnp.sum(lct * lct, axis=0, keepdims=True)
    hc4 = jnp.concatenate([hc, jnp.ones((ROWS, 1), jnp.float32)], axis=1)
    lct4 = jnp.concatenate([lct, -0.5 * ln], axis=0)
    g = lax.dot_general(hc4, lct4, (((1,), (0,)), ((), ())),
                        preferred_element_type=jnp.float32)
    idx = jnp.argmax(g, axis=1).astype(jnp.int32)
    idx_ref[...] = idx.reshape(ROWS, 1)


def _nn_indices(high_res_coord, lct):
    return pl.pallas_call(
        _argmin_body,
        grid=(NT,),
        in_specs=[
            pl.BlockSpec((ROWS, 3), lambda i: (i, 0)),
            pl.BlockSpec((3, M), lambda i: (0, 0)),
        ],
        out_specs=pl.BlockSpec((ROWS, 1), lambda i: (i, 0)),
        out_shape=jax.ShapeDtypeStruct((N, 1), jnp.int32),
    )(high_res_coord, lct)


def _hist_body(idx_hbm, out_hbm, idx_v, hist_v):
    c = lax.axis_index("c")
    s = lax.axis_index("s")
    wid = s * 2 + c
    base = wid * RPW
    pltpu.sync_copy(idx_hbm.at[pl.ds(base, RPW)], idx_v)
    zeros16 = jnp.zeros((LANES,), jnp.float32)

    def zero_body(i, carry):
        hist_v[pl.ds(i * LANES, LANES)] = zeros16
        return carry

    lax.fori_loop(0, M // LANES, zero_body, 0)

    ones16 = jnp.ones((LANES,), jnp.float32)
    lane_iota = lax.iota(jnp.int32, LANES)

    def scat_body(j, carry):
        idx16 = idx_v[pl.ds(j * LANES, LANES)]
        for l in range(LANES):
            plsc.addupdate_scatter(hist_v, [idx16], ones16,
                                   mask=lane_iota == l)
        return carry

    lax.fori_loop(0, RPW // LANES, scat_body, 0)
    pltpu.sync_copy(hist_v, out_hbm.at[wid])


def _partial_hist(idx_flat):
    mesh = plsc.VectorSubcoreMesh(core_axis_name="c", subcore_axis_name="s")
    return pl.kernel(
        _hist_body,
        out_type=jax.ShapeDtypeStruct((NW, M), jnp.float32),
        scratch_types=[
            pltpu.VMEM((RPW,), jnp.int32),
            pltpu.VMEM((M,), jnp.float32),
        ],
        mesh=mesh,
        compiler_params=pltpu.CompilerParams(needs_layout_passes=False),
    )(idx_flat)


FROWS = 4096
FNT = N // FROWS


def _fuse_body(ph_ref, lr_ref, hf_ref, hco_ref, off_ref,
               caw1_ref, cab1_ref, caw2_ref, cab2_ref,
               saw1_ref, sab1_ref, saw2_ref, sab2_ref,
               fmw_ref, fmb_ref, gam_ref, bet_ref, out_ref,
               y_s, cw_s, st_s):
    f32 = jnp.float32
    p = pl.program_id(0)
    t = pl.program_id(1)

    @pl.when(p == 0)
    def _phase0():
        @pl.when(t == 0)
        def _prep():
            ph = ph_ref[...]
            wpb = NW // B
            hist = jnp.concatenate(
                [jnp.sum(ph[wpb * b:wpb * (b + 1)], axis=0, keepdims=True)
                 for b in range(B)], axis=0)
            counts = jnp.sum(hist, axis=1, keepdims=True)
            gf = lax.dot_general(hist, lr_ref[...], (((1,), (0,)), ((), ())),
                                 preferred_element_type=f32)
            gf = gf / jnp.maximum(counts, 1.0)
            h = jnp.maximum(jnp.dot(gf, caw1_ref[...]) + cab1_ref[...], 0.0)
            cw_s[0:B] = jax.nn.sigmoid(jnp.dot(h, caw2_ref[...])
                                       + cab2_ref[...])
            st_s[...] = jnp.zeros((8, C), f32)

        cw = cw_s[0:B]
        ii = (t * FROWS
              + lax.broadcasted_iota(jnp.int32, (FROWS, B), 0))
        bidx = jnp.sum((ii >= off_ref[...]).astype(jnp.int32), axis=1,
                       keepdims=True)
        oh = (bidx == lax.broadcasted_iota(jnp.int32, (FROWS, B), 1))
        cwr = jnp.dot(oh.astype(f32), cw, preferred_element_type=f32)
        hf = hf_ref[...]
        cr = hf * cwr
        saw1 = saw1_ref[...]
        s1 = (jnp.dot(cr, saw1[:C]) + jnp.dot(hco_ref[...], saw1[C:])
              + sab1_ref[...])
        s1 = jnp.maximum(s1, 0.0)
        sw = jax.nn.sigmoid(jnp.dot(s1, saw2_ref[...]) + sab2_ref[...])
        sr = cr * sw
        fmw = fmw_ref[...]
        y = jnp.dot(hf, fmw[:C]) + jnp.dot(sr, fmw[C:]) + fmb_ref[...]
        y_s[pl.ds(t * FROWS, FROWS), :] = y
        st_s[0:1] += jnp.sum(y, axis=0, keepdims=True)
        st_s[1:2] += jnp.sum(y * y, axis=0, keepdims=True)

        @pl.when(t == FNT - 1)
        def _stats():
            mean = st_s[0:1] * (1.0 / N)
            var = st_s[1:2] * (1.0 / N) - mean * mean
            st_s[2:3] = mean
            st_s[3:4] = lax.rsqrt(var + 1e-5)

    @pl.when(p == 1)
    def _phase1():
        y = y_s[pl.ds(t * FROWS, FROWS), :]
        out_ref[...] = jnp.maximum(
            (y - st_s[2:3]) * st_s[3:4] * gam_ref[...] + bet_ref[...], 0.0)


def _fuse(ph, lr, hf, hco, off2d, caw1, cab1, caw2, cab2,
          saw1, sab1, saw2, sab2, fmw, fmb, gam, bet):
    full = lambda shape: pl.BlockSpec(shape, lambda p, t: tuple(
        0 for _ in shape))
    return pl.pallas_call(
        _fuse_body,
        grid=(2, FNT),
        in_specs=[
            full((NW, M)),
            full((M, C)),
            pl.BlockSpec((FROWS, C), lambda p, t: ((1 - p) * t, 0)),
            pl.BlockSpec((FROWS, 3), lambda p, t: ((1 - p) * t, 0)),
            full((1, B)),
            full((C, MID)), full((1, MID)), full((MID, C)), full((1, C)),
            full((C + 3, C)), full((1, C)), full((C, 1)), full((1, 1)),
            full((2 * C, C)), full((1, C)), full((1, C)), full((1, C)),
        ],
        out_specs=pl.BlockSpec((FROWS, C), lambda p, t: (p * t, 0)),
        out_shape=jax.ShapeDtypeStruct((N, C), jnp.float32),
        scratch_shapes=[
            pltpu.VMEM((N, C), jnp.float32),
            pltpu.VMEM((8, C), jnp.float32),
            pltpu.VMEM((8, C), jnp.float32),
        ],
    )(ph, lr, hf, hco, off2d, caw1, cab1, caw2, cab2,
      saw1, sab1, saw2, sab2, fmw, fmb, gam, bet)


@jax.jit
def kernel(high_res_feat, high_res_coord, low_res_feat, low_res_coord,
           high_res_offset, ca_w1, ca_b1, ca_w2, ca_b2,
           sa_w1, sa_b1, sa_w2, sa_b2, fm_w, fm_b, bn_gamma, bn_beta):
    lct = low_res_coord.T
    idx = _nn_indices(high_res_coord, lct)
    ph = _partial_hist(idx.reshape(N))
    ph = _partial_hist(idx.reshape(N))
    return _fuse(
        ph, low_res_feat, high_res_feat, high_res_coord,
        high_res_offset.reshape(1, B),
        ca_w1, ca_b1.reshape(1, -1), ca_w2, ca_b2.reshape(1, -1),
        sa_w1, sa_b1.reshape(1, -1), sa_w2, sa_b2.reshape(1, -1),
        fm_w, fm_b.reshape(1, -1), bn_gamma.reshape(1, -1),
        bn_beta.reshape(1, -1))
